# baseline (speedup 1.0000x reference)
import functools

import jax
import jax.numpy as jnp
import numpy as np
from jax import lax
from jax.experimental import pallas as pl
from jax.experimental.pallas import tpu as pltpu
from jax.experimental.pallas import tpu_sc as plsc

B = 16384
D = 64
BN_EPS = 1e-5

NC = 2
NS = 16
NW = NC * NS
BPW = B // NW
CH = 128
NCH = BPW // CH


@functools.lru_cache(maxsize=None)
def _make_sc_gather(nu, nm):
    mesh = plsc.VectorSubcoreMesh(core_axis_name="c", subcore_axis_name="s")

    @functools.partial(
        pl.kernel,
        mesh=mesh,
        compiler_params=pltpu.CompilerParams(needs_layout_passes=False),
        out_type=[
            jax.ShapeDtypeStruct((NW, D, BPW), jnp.float32),
            jax.ShapeDtypeStruct((NW, D, BPW), jnp.float32),
        ],
        scratch_types=[
            pltpu.VMEM((BPW,), jnp.int32),
            pltpu.VMEM((BPW,), jnp.int32),
            pltpu.VMEM((CH,), jnp.int32),
            pltpu.VMEM((CH,), jnp.int32),
            pltpu.VMEM((CH, 2 * D), jnp.float32),
            pltpu.VMEM((CH, 2 * D), jnp.float32),
            pltpu.VMEM((D, BPW), jnp.float32),
            pltpu.SemaphoreType.DMA,
            pltpu.SemaphoreType.DMA,
        ],
    )
    def _sc_gather(users_hbm, movies_hbm, ut_hbm, mt_hbm, ueT_hbm, meT_hbm,
                   idx_u, idx_m, ig0, ig1, rows0, rows1, out_T, sem0, sem1):
        wid = lax.axis_index("s") * NC + lax.axis_index("c")
        pltpu.sync_copy(users_hbm.at[wid], idx_u)
        pltpu.sync_copy(movies_hbm.at[wid], idx_m)
        igs = (ig0, ig1)
        rows = (rows0, rows1)
        sems = (sem0, sem1)
        lanes = lax.iota(jnp.int32, 16)

        def one_table(idx_ref, tbl_hbm, out_hbm):
            def fire(ch, buf):
                def fv(v, _):
                    igs[buf][pl.ds(v * 16, 16)] = lax.shift_right_logical(
                        idx_ref[pl.ds(ch * CH + v * 16, 16)], 1)
                    return 0

                lax.fori_loop(0, CH // 16, fv, 0)
                pltpu.async_copy(tbl_hbm.at[igs[buf]], rows[buf], sems[buf])

            def drain_select(ch, buf):
                pltpu.make_async_copy(tbl_hbm.at[igs[buf]], rows[buf],
                                      sems[buf]).wait()
                base = ch * CH

                def v_body(v, _):
                    iv = idx_ref[pl.ds(base + v * 16, 16)]
                    hv = lax.bitwise_and(iv, 1) * D
                    rowv = lanes + v * 16

                    def col_body(c, _):
                        colv = hv + jnp.full((16,), 0, jnp.int32) + c
                        vals = plsc.load_gather(rows[buf], [rowv, colv])
                        out_T[c, pl.ds(base + v * 16, 16)] = vals
                        return 0

                    lax.fori_loop(0, D, col_body, 0)
                    return 0

                lax.fori_loop(0, CH // 16, v_body, 0)

            fire(0, 0)
            for ch in range(NCH):
                if ch + 1 < NCH:
                    fire(ch + 1, (ch + 1) % 2)
                drain_select(ch, ch % 2)
            pltpu.sync_copy(out_T, out_hbm.at[wid])

        one_table(idx_u, ut_hbm, ueT_hbm)
        one_table(idx_m, mt_hbm, meT_hbm)

    return _sc_gather


def _mlp_body(ueT_ref, meT_ref, w1_ref, c1_ref, w2_ref, c2_ref, w3_ref,
              c3_ref, out_ref):
    w1 = w1_ref[...]
    tn = (((1,), (0,)), ((), ()))
    h = lax.dot_general(w1[:, :D], ueT_ref[0], tn,
                        preferred_element_type=jnp.float32)
    h += lax.dot_general(w1[:, D:], meT_ref[0], tn,
                         preferred_element_type=jnp.float32)
    h = jnp.maximum(h + c1_ref[...], 0.0)
    h = lax.dot_general(w2_ref[...], h, tn, preferred_element_type=jnp.float32)
    h = jnp.maximum(h + c2_ref[...], 0.0)
    o = jnp.sum(h * w3_ref[...], axis=0, keepdims=True) + c3_ref[...]
    out_ref[...] = 1.0 / (1.0 + jnp.exp(-o))


def kernel(users, movies, user_table, movie_table,
           W1, b1, g1, be1, W2, b2, g2, be2, W3, b3):
    u = users.astype(jnp.int32).reshape(NW, BPW)
    m = movies.astype(jnp.int32).reshape(NW, BPW)
    utp = user_table.reshape(user_table.shape[0] // 2, 2 * D)
    mtp = movie_table.reshape(movie_table.shape[0] // 2, 2 * D)
    ueT, meT = _make_sc_gather(utp.shape[0], mtp.shape[0])(u, m, utp, mtp)

    s = np.float32(1.0 / np.sqrt(1.0 + BN_EPS))
    w1 = W1 * (g1 * s)[:, None]
    c1 = (b1 * g1 * s + be1).reshape(32, 1)
    w2 = W2 * (g2 * s)[:, None]
    c2 = (b2 * g2 * s + be2).reshape(16, 1)
    w3 = W3.reshape(16, 1)
    c3 = b3.reshape(1, 1)

    outT = pl.pallas_call(
        _mlp_body,
        grid=(NW,),
        in_specs=[
            pl.BlockSpec((1, D, BPW), lambda w: (w, 0, 0)),
            pl.BlockSpec((1, D, BPW), lambda w: (w, 0, 0)),
            pl.BlockSpec((32, 128), lambda w: (0, 0)),
            pl.BlockSpec((32, 1), lambda w: (0, 0)),
            pl.BlockSpec((16, 32), lambda w: (0, 0)),
            pl.BlockSpec((16, 1), lambda w: (0, 0)),
            pl.BlockSpec((16, 1), lambda w: (0, 0)),
            pl.BlockSpec((1, 1), lambda w: (0, 0)),
        ],
        out_specs=pl.BlockSpec((1, BPW), lambda w: (0, w)),
        out_shape=jax.ShapeDtypeStruct((1, B), jnp.float32),
    )(ueT, meT, w1, c1, w2, c2, w3, c3)
    return outT.reshape(B, 1)
